# baseline (speedup 1.0000x reference)
import functools

import jax
import jax.numpy as jnp
from jax import lax
from jax.experimental import pallas as pl
from jax.experimental.pallas import tpu as pltpu
from jax.experimental.pallas import tpu_sc as plsc

DIM = 64
BATCH = 16384

_NC = 2
_NS = 16
_NW = _NC * _NS
_B_PER_W = BATCH // _NW
_CHUNK = 32
_N_CHUNK = _B_PER_W // _CHUNK

_mesh = plsc.VectorSubcoreMesh(core_axis_name="c", subcore_axis_name="s")


@functools.partial(
    pl.kernel,
    mesh=_mesh,
    out_type=jax.ShapeDtypeStruct((BATCH * DIM,), jnp.float32),
    scratch_types=[
        pltpu.VMEM((_B_PER_W,), jnp.int32),
        pltpu.VMEM((_CHUNK * DIM,), jnp.float32),
        pltpu.SemaphoreType.DMA,
    ],
)
def _gather_rows(idx_hbm, tab_hbm, out_hbm, idx_v, sel_v, sem):
    wid = lax.axis_index("s") * _NC + lax.axis_index("c")
    base = wid * _B_PER_W
    pltpu.sync_copy(idx_hbm.at[pl.ds(base, _B_PER_W)], idx_v)

    def chunk_body(c, _):
        copies = []
        for g in range(_CHUNK // 16):
            vec = idx_v[pl.ds(c * _CHUNK + g * 16, 16)] * DIM
            for i in range(16):
                off = pl.multiple_of(
                    lax.squeeze(lax.slice(vec, (i,), (i + 1,)), (0,)), DIM
                )
                copies.append(
                    pltpu.async_copy(
                        tab_hbm.at[pl.ds(off, DIM)],
                        sel_v.at[pl.ds((g * 16 + i) * DIM, DIM)],
                        sem,
                    )
                )
        for cp in copies:
            cp.wait()
        pltpu.sync_copy(
            sel_v,
            out_hbm.at[pl.ds((base + c * _CHUNK) * DIM, _CHUNK * DIM)],
        )
        return ()

    lax.fori_loop(0, _N_CHUNK, chunk_body, (), unroll=False)


def kernel(ind, z):
    if ind.ndim == 0:
        ind = ind.reshape((1,))
    out = _gather_rows(ind, z.reshape(-1))
    return out.reshape(ind.shape[0], 1, DIM)


# speedup vs baseline: 2.3839x; 2.3839x over previous
import functools

import jax
import jax.numpy as jnp
from jax import lax
from jax.experimental import pallas as pl
from jax.experimental.pallas import tpu as pltpu
from jax.experimental.pallas import tpu_sc as plsc

DIM = 64
BATCH = 16384
GRP = 8
N_GRP = 125000

_NC = 2
_NS = 16
_NW = _NC * _NS
_B_PER_W = BATCH // _NW
_CHUNK = 32
_N_CHUNK = _B_PER_W // _CHUNK

_mesh = plsc.VectorSubcoreMesh(core_axis_name="c", subcore_axis_name="s")


@functools.partial(
    pl.kernel,
    mesh=_mesh,
    out_type=jax.ShapeDtypeStruct((BATCH, 1, DIM), jnp.float32),
    scratch_types=[
        pltpu.VMEM((_B_PER_W,), jnp.int32),
        pltpu.VMEM((_CHUNK, DIM), jnp.float32),
        pltpu.SemaphoreType.DMA,
    ],
)
def _gather_rows(idx_hbm, tab_hbm, out_hbm, idx_v, sel_v, sem):
    wid = lax.axis_index("s") * _NC + lax.axis_index("c")
    base = wid * _B_PER_W
    pltpu.sync_copy(idx_hbm.at[pl.ds(base, _B_PER_W)], idx_v)

    def chunk_body(c, _):
        copies = []
        for g in range(_CHUNK // 16):
            vec = idx_v[pl.ds(c * _CHUNK + g * 16, 16)]
            gv = lax.shift_right_logical(vec, 3)
            sv = jnp.bitwise_and(vec, 7)
            for i in range(16):
                gi = lax.squeeze(lax.slice(gv, (i,), (i + 1,)), (0,))
                si = lax.squeeze(lax.slice(sv, (i,), (i + 1,)), (0,))
                copies.append(
                    pltpu.async_copy(
                        tab_hbm.at[gi, si],
                        sel_v.at[g * 16 + i],
                        sem,
                    )
                )
        for cp in copies:
            cp.wait()
        pltpu.sync_copy(
            sel_v, out_hbm.at[pl.ds(base + c * _CHUNK, _CHUNK), 0, :]
        )
        return ()

    lax.fori_loop(0, _N_CHUNK, chunk_body, (), unroll=False)


def kernel(ind, z):
    if ind.ndim == 0:
        ind = ind.reshape((1,))
    z3 = z.reshape(N_GRP, GRP, DIM)
    return _gather_rows(ind, z3)


# speedup vs baseline: 2.4818x; 1.0410x over previous
import functools

import jax
import jax.numpy as jnp
from jax import lax
from jax.experimental import pallas as pl
from jax.experimental.pallas import tpu as pltpu
from jax.experimental.pallas import tpu_sc as plsc

DIM = 64
BATCH = 16384
GRP = 8
N_GRP = 125000

_NC = 2
_NS = 16
_NW = _NC * _NS
_B_PER_W = BATCH // _NW
_CHUNK = 32
_N_CHUNK = _B_PER_W // _CHUNK

_mesh = plsc.VectorSubcoreMesh(core_axis_name="c", subcore_axis_name="s")


@functools.partial(
    pl.kernel,
    mesh=_mesh,
    out_type=jax.ShapeDtypeStruct((BATCH, DIM), jnp.float32),
    scratch_types=[
        pltpu.VMEM((_B_PER_W,), jnp.int32),
        pltpu.VMEM((_CHUNK, DIM), jnp.float32),
        pltpu.SemaphoreType.DMA,
    ],
)
def _gather_rows(idx_hbm, tab_hbm, out_hbm, idx_v, sel_v, sem):
    wid = lax.axis_index("s") * _NC + lax.axis_index("c")
    base = wid * _B_PER_W
    pltpu.sync_copy(idx_hbm.at[pl.ds(base, _B_PER_W)], idx_v)

    def chunk_body(c, _):
        copies = []
        for g in range(_CHUNK // 16):
            vec = idx_v[pl.ds(c * _CHUNK + g * 16, 16)]
            gv = lax.shift_right_logical(vec, 3)
            sv = jnp.bitwise_and(vec, 7)
            for i in range(16):
                gi = lax.squeeze(lax.slice(gv, (i,), (i + 1,)), (0,))
                si = lax.squeeze(lax.slice(sv, (i,), (i + 1,)), (0,))
                copies.append(
                    pltpu.async_copy(
                        tab_hbm.at[gi, si],
                        sel_v.at[g * 16 + i],
                        sem,
                    )
                )
        for cp in copies:
            cp.wait()
        pltpu.sync_copy(
            sel_v, out_hbm.at[pl.ds(base + c * _CHUNK, _CHUNK), :]
        )
        return ()

    lax.fori_loop(0, _N_CHUNK, chunk_body, (), unroll=False)


def kernel(ind, z):
    if ind.ndim == 0:
        ind = ind.reshape((1,))
    z3 = z.reshape(N_GRP, GRP, DIM)
    out = _gather_rows(ind, z3)
    return out.reshape(ind.shape[0], 1, DIM)
